# Initial kernel scaffold; baseline (speedup 1.0000x reference)
import jax
import jax.numpy as jnp
from jax.experimental import pallas as pl


def kernel(x, edge_index, W_conv, b_conv, ln_g, ln_b, W_fc, b_fc):
    raise NotImplementedError("write your pallas kernel here")


# speedup vs baseline: 13.5498x; 13.5498x over previous
import functools

import jax
import jax.numpy as jnp
from jax import lax
from jax.experimental import pallas as pl
from jax.experimental.pallas import tpu as pltpu
from jax.experimental.pallas import tpu_sc as plsc

N = 10000
E = 320000
D = 128
C = 64

NC = 2
NS = 16
NW = NC * NS
EPT = E // NW
CH = 80
NCH = EPT // CH

DEG_W = 128
DEG_PAD = 10240
DEG_ROWS = DEG_PAD // NS
ACC_PAD = 10240
ACC_ROWS = ACC_PAD // NS

ROW_BLK = 1000
GRID = N // ROW_BLK


def _mesh():
    return plsc.VectorSubcoreMesh(core_axis_name="c", subcore_axis_name="s")


def _sc_degree(dst, ones_h, zeros_h):
    @functools.partial(
        pl.kernel,
        mesh=_mesh(),
        out_type=jax.ShapeDtypeStruct((2 * DEG_PAD, DEG_W), jnp.float32),
        scratch_types=[
            pltpu.VMEM((CH,), jnp.int32),
            pltpu.VMEM((CH, DEG_W), jnp.float32),
            pltpu.VMEM_SHARED((DEG_PAD, DEG_W), jnp.float32),
        ],
    )
    def k(dst_hbm, ones_hbm, zeros_hbm, out_hbm, idx_v, ones_v, deg_sh):
        c = lax.axis_index("c")
        s = lax.axis_index("s")
        wid = s * NC + c
        pltpu.sync_copy(zeros_hbm.at[pl.ds(s * DEG_ROWS, DEG_ROWS)],
                        deg_sh.at[pl.ds(s * DEG_ROWS, DEG_ROWS)])
        pltpu.sync_copy(ones_hbm, ones_v)
        plsc.subcore_barrier()

        base = wid * EPT

        def body(i, carry):
            off = pl.multiple_of(base + i * CH, 8)
            pltpu.sync_copy(dst_hbm.at[pl.ds(off, CH)], idx_v)
            pltpu.sync_copy(ones_v, deg_sh.at[idx_v], add=True)
            return carry

        lax.fori_loop(0, NCH, body, 0)
        plsc.subcore_barrier()
        pltpu.sync_copy(deg_sh.at[pl.ds(s * DEG_ROWS, DEG_ROWS)],
                        out_hbm.at[pl.ds(c * DEG_PAD + s * DEG_ROWS, DEG_ROWS)])

    return k(dst, ones_h, zeros_h)


def _sc_scatter(src, dst, g, zeros2d):
    @functools.partial(
        pl.kernel,
        mesh=_mesh(),
        out_type=jax.ShapeDtypeStruct((2 * ACC_PAD, D), jnp.float32),
        scratch_types=[
            pltpu.VMEM((CH,), jnp.int32),
            pltpu.VMEM((CH,), jnp.int32),
            pltpu.VMEM((CH, D), jnp.float32),
            pltpu.VMEM_SHARED((ACC_PAD, D), jnp.float32),
            pltpu.SemaphoreType.DMA,
        ],
    )
    def k(src_hbm, dst_hbm, g_hbm, zeros_hbm, out_hbm, sidx, didx, rows, acc, sem):
        c = lax.axis_index("c")
        s = lax.axis_index("s")
        wid = s * NC + c
        pltpu.sync_copy(zeros_hbm.at[pl.ds(s * ACC_ROWS, ACC_ROWS)],
                        acc.at[pl.ds(s * ACC_ROWS, ACC_ROWS)])
        plsc.subcore_barrier()

        base = wid * EPT

        def body(i, carry):
            off = pl.multiple_of(base + i * CH, 8)
            pltpu.sync_copy(src_hbm.at[pl.ds(off, CH)], sidx)
            pltpu.sync_copy(dst_hbm.at[pl.ds(off, CH)], didx)
            pltpu.async_copy(g_hbm.at[sidx], rows, sem).wait()
            pltpu.sync_copy(rows, acc.at[didx], add=True)
            return carry

        lax.fori_loop(0, NCH, body, 0)
        plsc.subcore_barrier()
        pltpu.sync_copy(acc.at[pl.ds(s * ACC_ROWS, ACC_ROWS)],
                        out_hbm.at[pl.ds(c * ACC_PAD + s * ACC_ROWS, ACC_ROWS)])

    return k(src, dst, g, zeros2d)


def _tc_g(x, w, dga, dgb):
    def body(x_ref, w_ref, a_ref, b_ref, o_ref):
        dis = lax.rsqrt(a_ref[...] + b_ref[...] + 1.0)
        h = jnp.dot(x_ref[...], w_ref[...], preferred_element_type=jnp.float32)
        o_ref[...] = h * dis

    return pl.pallas_call(
        body,
        grid=(GRID,),
        in_specs=[
            pl.BlockSpec((ROW_BLK, D), lambda i: (i, 0)),
            pl.BlockSpec((D, D), lambda i: (0, 0)),
            pl.BlockSpec((ROW_BLK, 1), lambda i: (i, 0)),
            pl.BlockSpec((ROW_BLK, 1), lambda i: (i, 0)),
        ],
        out_specs=pl.BlockSpec((ROW_BLK, D), lambda i: (i, 0)),
        out_shape=jax.ShapeDtypeStruct((N, D), jnp.float32),
    )(x, w, dga, dgb)


def _tc_final(acc_a, acc_b, g, x, dga, dgb, b_conv, ln_g, ln_b, w_fc, b_fc):
    def body(a_ref, b_ref, g_ref, x_ref, da_ref, db_ref, bc_ref, lg_ref,
             lb_ref, wf_ref, bf_ref, o_ref):
        dis = lax.rsqrt(da_ref[...] + db_ref[...] + 1.0)
        conv = (a_ref[...] + b_ref[...] + g_ref[...]) * dis + bc_ref[...]
        mu = jnp.mean(conv, axis=-1, keepdims=True)
        cen = conv - mu
        var = jnp.mean(cen * cen, axis=-1, keepdims=True)
        ln = cen * lax.rsqrt(var + 1e-5) * lg_ref[...] + lb_ref[...]
        h = jnp.maximum(ln, 0.0) * x_ref[...]
        o_ref[...] = (jnp.dot(h, wf_ref[...], preferred_element_type=jnp.float32)
                      + bf_ref[...])

    return pl.pallas_call(
        body,
        grid=(GRID,),
        in_specs=[
            pl.BlockSpec((ROW_BLK, D), lambda i: (i, 0)),
            pl.BlockSpec((ROW_BLK, D), lambda i: (i, 0)),
            pl.BlockSpec((ROW_BLK, D), lambda i: (i, 0)),
            pl.BlockSpec((ROW_BLK, D), lambda i: (i, 0)),
            pl.BlockSpec((ROW_BLK, 1), lambda i: (i, 0)),
            pl.BlockSpec((ROW_BLK, 1), lambda i: (i, 0)),
            pl.BlockSpec((1, D), lambda i: (0, 0)),
            pl.BlockSpec((1, D), lambda i: (0, 0)),
            pl.BlockSpec((1, D), lambda i: (0, 0)),
            pl.BlockSpec((D, C), lambda i: (0, 0)),
            pl.BlockSpec((1, C), lambda i: (0, 0)),
        ],
        out_specs=pl.BlockSpec((ROW_BLK, C), lambda i: (i, 0)),
        out_shape=jax.ShapeDtypeStruct((N, C), jnp.float32),
    )(acc_a, acc_b, g, x, dga, dgb, b_conv, ln_g, ln_b, w_fc, b_fc)


def kernel(x, edge_index, W_conv, b_conv, ln_g, ln_b, W_fc, b_fc):
    ei = edge_index.astype(jnp.int32)
    src = ei[0]
    dst = ei[1]

    zeros_acc = jnp.zeros((ACC_PAD, D), jnp.float32)
    ones_h = jnp.ones((CH, DEG_W), jnp.float32)
    deg2 = _sc_degree(dst, ones_h, zeros_acc)
    dga = deg2[:N, 0:1]
    dgb = deg2[DEG_PAD:DEG_PAD + N, 0:1]

    g = _tc_g(x, W_conv, dga, dgb)

    acc2 = _sc_scatter(src, dst, g, zeros_acc)

    return _tc_final(acc2[:N], acc2[ACC_PAD:ACC_PAD + N], g, x, dga, dgb,
                     b_conv.reshape(1, D), ln_g.reshape(1, D),
                     ln_b.reshape(1, D), W_fc, b_fc.reshape(1, C))
